# baseline (speedup 1.0000x reference)
import functools

import jax
import jax.numpy as jnp
from jax import lax
from jax.experimental import pallas as pl
from jax.experimental.pallas import tpu as pltpu
from jax.experimental.pallas import tpu_sc as plsc

VOCAB = 100000
HID = 128
B = 1024
S = 200
HS = S // 2
EPS = 1e-12

NW = 32
NSLAB = 4
SB = B // NSLAB
STOK = SB * S
TOK_PER_W = STOK // NW
CHUNK = 80
NCHUNK = TOK_PER_W // CHUNK
NPAIR = CHUNK // 2
BB = 16
SLAB_STEPS = SB // BB

_LO16 = 0xFFFF
_HI16 = -65536


def _pack_chunk(buf, pk):
    def body(r, carry):
        for w in range(HID // 16):
            a = buf[2 * r, pl.ds(16 * w, 16)]
            b = buf[2 * r + 1, pl.ds(16 * w, 16)]
            pk[r, pl.ds(16 * w, 16)] = (
                ((a >> 16) & _LO16) | (b & _HI16))
        return carry

    lax.fori_loop(0, NPAIR, body, 0)


def _gather_kernel(ids_hbm, table_hbm, out_hbm, idx_v, buf0, buf1, pk,
                   sem0, sem1):
    wid = lax.axis_index("s") * 2 + lax.axis_index("c")
    pair_base = wid * (TOK_PER_W // 2)
    pltpu.sync_copy(ids_hbm.at[wid], idx_v)
    bufs = (buf0, buf1)
    sems = (sem0, sem1)

    def start(c):
        return pltpu.async_copy(
            table_hbm.at[idx_v.at[pl.ds(c * CHUNK, CHUNK)]],
            bufs[c % 2], sems[c % 2])

    handles = [None] * NCHUNK
    handles[0] = start(0)
    for c in range(NCHUNK):
        if c + 1 < NCHUNK:
            handles[c + 1] = start(c + 1)
        handles[c].wait()
        _pack_chunk(bufs[c % 2], pk)
        pltpu.sync_copy(pk, out_hbm.at[pl.ds(pair_base + c * NPAIR, NPAIR)])


def _sc_gather(ids, table_i32):
    mesh = plsc.VectorSubcoreMesh(core_axis_name="c", subcore_axis_name="s")
    k = functools.partial(
        pl.kernel,
        mesh=mesh,
        out_type=jax.ShapeDtypeStruct((STOK // 2, HID), jnp.int32),
        scratch_types=[
            pltpu.VMEM((TOK_PER_W,), jnp.int32),
            pltpu.VMEM((CHUNK, HID), jnp.int32),
            pltpu.VMEM((CHUNK, HID), jnp.int32),
            pltpu.VMEM((NPAIR, HID), jnp.int32),
            pltpu.SemaphoreType.DMA,
            pltpu.SemaphoreType.DMA,
        ],
    )(_gather_kernel)
    return k(ids, table_i32)


def _plane_ln(x, posp, ttp, t0, t1, gamma, beta):
    emb = (x + posp[None, :, :] + t0[None, None, :]
           + ttp[:, :, None] * (t1 - t0)[None, None, :])
    mu = jnp.mean(emb, axis=-1, keepdims=True)
    xc = emb - mu
    var = jnp.mean(xc * xc, axis=-1, keepdims=True)
    y = xc * lax.rsqrt(var + EPS)
    return y * gamma[None, None, :] + beta[None, None, :]


def _ln_kernel(words_ref, tte_ref, tto_ref, pose_ref, poso_ref, type_ref,
               gamma_ref, beta_ref, prev_ref, out_ref):
    del prev_ref
    u = words_ref[...]
    xe = lax.bitcast_convert_type(u << 16, jnp.float32)
    xo = lax.bitcast_convert_type(u & _HI16, jnp.float32)
    tte = tte_ref[:, 0, :].astype(jnp.float32)
    tto = tto_ref[:, 0, :].astype(jnp.float32)
    t0 = type_ref[0]
    t1 = type_ref[1]
    g = gamma_ref[0]
    bt = beta_ref[0]
    out_ref[:, :, 0, :] = _plane_ln(xe, pose_ref[...], tte, t0, t1, g, bt)
    out_ref[:, :, 1, :] = _plane_ln(xo, poso_ref[...], tto, t0, t1, g, bt)


def _tc_add_ln(slab, words, tte, tto, pose, poso, type_emb, gamma, beta, prev):
    in_specs = [
        pl.BlockSpec((BB, HS, HID), lambda i: (i, 0, 0)),
        pl.BlockSpec((BB, 1, HS), lambda i: (i, 0, 0)),
        pl.BlockSpec((BB, 1, HS), lambda i: (i, 0, 0)),
        pl.BlockSpec((HS, HID), lambda i: (0, 0)),
        pl.BlockSpec((HS, HID), lambda i: (0, 0)),
        pl.BlockSpec((2, HID), lambda i: (0, 0)),
        pl.BlockSpec((1, HID), lambda i: (0, 0)),
        pl.BlockSpec((1, HID), lambda i: (0, 0)),
    ]
    args = [words, tte, tto, pose, poso, type_emb, gamma, beta]
    aliases = {}
    body = _ln_kernel
    if prev is not None:
        in_specs.append(pl.BlockSpec(memory_space=pl.ANY))
        args.append(prev)
        aliases = {8: 0}
    else:
        body = functools.partial(
            lambda *refs: _ln_kernel(*refs[:8], None, refs[8]))
    return pl.pallas_call(
        body,
        grid=(SLAB_STEPS,),
        in_specs=in_specs,
        out_specs=pl.BlockSpec(
            (BB, HS, 2, HID), lambda i, _s=slab: (_s * SLAB_STEPS + i, 0, 0, 0)),
        out_shape=jax.ShapeDtypeStruct((B, HS, 2, HID), jnp.float32),
        input_output_aliases=aliases,
    )(*args)


def kernel(input_ids, token_type_ids, word_emb, pos_emb, type_emb, gamma, beta):
    ids = input_ids.astype(jnp.int32).reshape(NSLAB, NW, TOK_PER_W)
    table_i32 = lax.bitcast_convert_type(word_emb, jnp.int32)
    tt = token_type_ids.astype(jnp.int32).reshape(NSLAB, SB, S)
    tte = tt[:, :, 0::2].reshape(NSLAB, SB, 1, HS)
    tto = tt[:, :, 1::2].reshape(NSLAB, SB, 1, HS)
    pose = pos_emb[0:S:2]
    poso = pos_emb[1:S:2]
    g2 = gamma.reshape(1, HID)
    b2 = beta.reshape(1, HID)
    slab_words = [
        _sc_gather(ids[s], table_i32).reshape(SB, HS, HID) for s in range(NSLAB)
    ]
    out = None
    for s in range(NSLAB):
        out = _tc_add_ln(s, slab_words[s], tte[s], tto[s], pose, poso,
                         type_emb, g2, b2, out)
    return out.reshape(B, S, HID)


# speedup vs baseline: 1.6803x; 1.6803x over previous
import functools

import jax
import jax.numpy as jnp
from jax import lax
from jax.experimental import pallas as pl
from jax.experimental.pallas import tpu as pltpu
from jax.experimental.pallas import tpu_sc as plsc

VOCAB = 100000
HID = 128
B = 1024
S = 200
EPS = 1e-12

NW = 32
NSLAB = 2
SB = B // NSLAB
STOK = SB * S
TOK_PER_W = STOK // NW
CHUNK = 128
NCHUNK = TOK_PER_W // CHUNK
BB = 32
SLAB_STEPS = SB // BB


def _gather_kernel(ids_hbm, table_hbm, out_hbm, idx_v, buf0, buf1, sem0, sem1):
    wid = lax.axis_index("s") * 2 + lax.axis_index("c")
    base = wid * TOK_PER_W
    pltpu.sync_copy(ids_hbm.at[wid], idx_v)
    bufs = (buf0, buf1)
    sems = (sem0, sem1)

    def start(c):
        return pltpu.async_copy(
            table_hbm.at[idx_v.at[pl.ds(c * CHUNK, CHUNK)]],
            bufs[c % 2], sems[c % 2])

    handles = [None] * NCHUNK
    handles[0] = start(0)
    for c in range(NCHUNK):
        if c + 1 < NCHUNK:
            handles[c + 1] = start(c + 1)
        handles[c].wait()
        pltpu.sync_copy(bufs[c % 2], out_hbm.at[pl.ds(base + c * CHUNK, CHUNK)])


def _sc_gather(ids, word_emb):
    mesh = plsc.VectorSubcoreMesh(core_axis_name="c", subcore_axis_name="s")
    k = functools.partial(
        pl.kernel,
        mesh=mesh,
        out_type=jax.ShapeDtypeStruct((STOK, HID), jnp.float32),
        scratch_types=[
            pltpu.VMEM((TOK_PER_W,), jnp.int32),
            pltpu.VMEM((CHUNK, HID), jnp.float32),
            pltpu.VMEM((CHUNK, HID), jnp.float32),
            pltpu.SemaphoreType.DMA,
            pltpu.SemaphoreType.DMA,
        ],
    )(_gather_kernel)
    return k(ids, word_emb)


def _ln_kernel(words_ref, tt_ref, pt0_ref, ptd_ref, gamma_ref, beta_ref,
               prev_ref, out_ref):
    del prev_ref
    words = words_ref[...]
    tt = tt_ref[:, 0, :].astype(jnp.float32)
    emb = (words + pt0_ref[...][None, :, :]
           + tt[:, :, None] * ptd_ref[0][None, None, :])
    mu = jnp.mean(emb, axis=-1, keepdims=True)
    xc = emb - mu
    var = jnp.mean(xc * xc, axis=-1, keepdims=True)
    y = xc * lax.rsqrt(var + EPS)
    out_ref[...] = y * gamma_ref[0][None, None, :] + beta_ref[0][None, None, :]


def _tc_add_ln(slab, words, token_type_ids, pt0, ptd, gamma, beta, prev):
    in_specs = [
        pl.BlockSpec((BB, S, HID), lambda i: (i, 0, 0)),
        pl.BlockSpec((BB, 1, S), lambda i: (i, 0, 0)),
        pl.BlockSpec((S, HID), lambda i: (0, 0)),
        pl.BlockSpec((1, HID), lambda i: (0, 0)),
        pl.BlockSpec((1, HID), lambda i: (0, 0)),
        pl.BlockSpec((1, HID), lambda i: (0, 0)),
    ]
    args = [words, token_type_ids, pt0, ptd, gamma, beta]
    aliases = {}
    body = _ln_kernel
    if prev is not None:
        in_specs.append(pl.BlockSpec(memory_space=pl.ANY))
        args.append(prev)
        aliases = {6: 0}
    else:
        body = functools.partial(
            lambda *refs: _ln_kernel(*refs[:6], None, refs[6]))
    return pl.pallas_call(
        body,
        grid=(SLAB_STEPS,),
        in_specs=in_specs,
        out_specs=pl.BlockSpec(
            (BB, S, HID), lambda i, _s=slab: (_s * SLAB_STEPS + i, 0, 0)),
        out_shape=jax.ShapeDtypeStruct((B, S, HID), jnp.float32),
        input_output_aliases=aliases,
    )(*args)


def kernel(input_ids, token_type_ids, word_emb, pos_emb, type_emb, gamma, beta):
    ids = input_ids.astype(jnp.int32).reshape(NSLAB, NW, TOK_PER_W)
    tt = token_type_ids.astype(jnp.int32).reshape(NSLAB, SB, 1, S)
    pt0 = pos_emb[:S] + type_emb[0][None, :]
    ptd = (type_emb[1] - type_emb[0]).reshape(1, HID)
    g2 = gamma.reshape(1, HID)
    b2 = beta.reshape(1, HID)
    slab_words = [
        _sc_gather(ids[s], word_emb).reshape(SB, S, HID) for s in range(NSLAB)
    ]
    out = None
    for s in range(NSLAB):
        out = _tc_add_ln(s, slab_words[s], tt[s], pt0, ptd, g2, b2, out)
    return out.reshape(B, S, HID)
